# baseline (speedup 1.0000x reference)
import functools

import jax
import jax.numpy as jnp
from jax import lax
from jax.experimental import pallas as pl
from jax.experimental.pallas import tpu as pltpu
from jax.experimental.pallas import tpu_sc as plsc

B, K, L = 128, 36, 20
WORD_DIM = 300
QUES_DIM = 1024
V_DIM = 2048
SCORE_DIM = 1024
HID = 100


def _leaky(x):
    return jnp.maximum(x, 0.01 * x)


@functools.lru_cache(maxsize=None)
def _make_sc_gather(V, D, N):
    NC, NS = 2, 16
    NW = NC * NS
    n_per_w = N // NW
    mesh = plsc.VectorSubcoreMesh(core_axis_name="c", subcore_axis_name="s",
                                  num_cores=NC, num_subcores=NS)

    @functools.partial(
        pl.kernel,
        out_type=jax.ShapeDtypeStruct((N, D), jnp.float32),
        mesh=mesh,
        scratch_types=[
            pltpu.VMEM((B,), jnp.int32),
            pltpu.VMEM((B, D), jnp.float32),
            pltpu.SemaphoreType.DMA,
        ],
    )
    def gather(table_hbm, idxt_hbm, out_hbm, idx_v, rows_v, sem):
        wid = lax.axis_index("s") * NC + lax.axis_index("c")

        @pl.when(wid < L)
        def _():
            pltpu.sync_copy(idxt_hbm.at[wid], idx_v)
            pltpu.async_copy(table_hbm.at[idx_v], rows_v, sem).wait()
            pltpu.sync_copy(rows_v, out_hbm.at[pl.ds(wid * B, B)])

    return gather


_PAD_ROWS = 2048


def _pad_body(in_ref, q_ref, out_ref, idxt_ref):
    blk = in_ref.shape[0]
    out_ref[...] = jnp.concatenate(
        [in_ref[...], jnp.zeros((blk, 384 - WORD_DIM), jnp.float32)], axis=1)

    @pl.when(pl.program_id(0) == 0)
    def _():
        idxt_ref[...] = jnp.transpose(q_ref[...])


def _pad_call(table, q):
    V = table.shape[0]
    grid = (pl.cdiv(V, _PAD_ROWS),)
    return pl.pallas_call(
        _pad_body,
        grid=grid,
        in_specs=[pl.BlockSpec((_PAD_ROWS, WORD_DIM), lambda i: (i, 0)),
                  pl.BlockSpec((B, L), lambda i: (0, 0))],
        out_specs=[pl.BlockSpec((_PAD_ROWS, 384), lambda i: (i, 0)),
                   pl.BlockSpec((L, B), lambda i: (0, 0))],
        out_shape=[jax.ShapeDtypeStruct((V, 384), jnp.float32),
                   jax.ShapeDtypeStruct((L, B), jnp.int32)],
    )(table, q)


_CHUNK = L // 4


def _gru_body(words_ref, Wih_ref, Whh_ref, bih_ref, bhh_ref, Wqp_ref, bqp_ref,
              Ws_ref, Wqr_ref, bqr_ref, qz_ref, wq_ref, gi_ref):
    Wih = Wih_ref[...]
    Whh = Whh_ref[...]
    bih = bih_ref[...]
    bhh = bhh_ref[...]

    def step(t, h):
        gi = gi_ref[pl.ds(t * B, B), :]
        gh = lax.dot_general(h, Whh, (((1,), (1,)), ((), ())),
                             preferred_element_type=jnp.float32) + bhh
        r = jax.nn.sigmoid(gi[:, :QUES_DIM] + gh[:, :QUES_DIM])
        z = jax.nn.sigmoid(gi[:, QUES_DIM:2 * QUES_DIM] + gh[:, QUES_DIM:2 * QUES_DIM])
        n = jnp.tanh(gi[:, 2 * QUES_DIM:] + r * gh[:, 2 * QUES_DIM:])
        return (1.0 - z) * n + z * h

    h = jnp.zeros((B, QUES_DIM), jnp.float32)
    for part in range(4):
        xs = words_ref[pl.ds(part * _CHUNK * B, _CHUNK * B), :WORD_DIM]
        gi_ref[...] = lax.dot_general(xs, Wih, (((1,), (1,)), ((), ())),
                                      preferred_element_type=jnp.float32) + bih
        h = lax.fori_loop(0, _CHUNK, step, h)

    qp = _leaky(lax.dot_general(h, Wqp_ref[...], (((1,), (1,)), ((), ())),
                                preferred_element_type=jnp.float32) + bqp_ref[...])
    qz_ref[...] = qp * Ws_ref[...]
    wq_ref[...] = jnp.sum(h * Wqr_ref[...], axis=1, keepdims=True) + bqr_ref[...]


def _gru_call(words2d, W_ih, W_hh, b_ih, b_hh, Wqp, bqp, Ws, Wqr, bqr):
    return pl.pallas_call(
        _gru_body,
        out_shape=(
            jax.ShapeDtypeStruct((B, SCORE_DIM), jnp.float32),
            jax.ShapeDtypeStruct((B, 1), jnp.float32),
        ),
        scratch_shapes=[pltpu.VMEM((_CHUNK * B, 3 * QUES_DIM), jnp.float32)],
    )(words2d, W_ih, W_hh, b_ih, b_hh, Wqp, bqp, Ws, Wqr, bqr)


def _head_body(bk, v2_ref, b3_ref, qz_ref, wq_ref, Wv_ref, bv_ref,
               WfJ_ref, WfI_ref, Wrows_ref, bs_ref, bd_ref,
               kappa_ref, rho_ref, G_ref, vtv_ref, km_ref):
    v2 = v2_ref[...]
    vp = _leaky(lax.dot_general(v2, Wv_ref[...], (((1,), (1,)), ((), ())),
                                preferred_element_type=jnp.float32) + bv_ref[...])
    qz = qz_ref[...]
    km_ref[...] = lax.dot_general(vp, qz, (((1,), (1,)), ((), ())),
                                  preferred_element_type=jnp.float32)
    kappa_ref[...] = jnp.concatenate(
        [jnp.transpose(km_ref[pl.ds(s * K, K), s:s + 1]) for s in range(bk)],
        axis=0) + bs_ref[0]

    ssq = jnp.sum(v2 * v2, axis=1, keepdims=True)
    inv = 1.0 / jnp.maximum(jnp.sqrt(ssq), 1e-12)
    nv = v2 * inv
    G_ref[...] = lax.dot_general(nv, nv, (((1,), (1,)), ((), ())),
                                 preferred_element_type=jnp.float32)

    Wf0 = Wrows_ref[0:1, :]
    Wf1 = Wrows_ref[1:2, :]
    Wf14 = Wrows_ref[2:3, :]
    Wf15 = Wrows_ref[3:4, :]
    Wf16 = Wrows_ref[4:5, :]
    bf = Wrows_ref[5:6, :]
    Wd0 = Wrows_ref[6:7, :]
    bd = bd_ref[0]

    for s in range(bk):
        vtv_ref[...] = G_ref[pl.ds(s * K, K), pl.ds(s * K, K)]
        vtv = vtv_ref[...]

        boxes = b3_ref[s]
        x1 = boxes[:, 0:1]
        y1 = boxes[:, 1:2]
        x2 = boxes[:, 2:3]
        y2 = boxes[:, 3:4]
        area = (x2 - x1) * (y2 - y1)

        def bi(col):
            return jnp.broadcast_to(col, (K, K))

        def bj(col):
            return jnp.broadcast_to(jnp.transpose(col), (K, K))

        rl = jnp.maximum(bj(x1), bi(x1))
        dt = jnp.maximum(bj(y1), bi(y1))
        lr = jnp.minimum(bj(x2), bi(x2))
        td = jnp.minimum(bj(y2), bi(y2))
        olap = jnp.maximum(0.0, lr - rl) * jnp.maximum(0.0, td - dt)
        area_i = bi(area)
        area_j = bj(area)
        iou = olap / (area_i + area_j - olap)
        o_ij = olap / area_j
        o_ji = olap / area_i

        fj = lax.dot_general(boxes, WfJ_ref[...], (((1,), (1,)), ((), ())),
                             preferred_element_type=jnp.float32)
        fi = lax.dot_general(boxes, WfI_ref[...], (((1,), (1,)), ((), ())),
                             preferred_element_type=jnp.float32)
        wq_s = wq_ref[s:s + 1, :]
        fjc = fj + wq_s * Wf0 + bf

        base = (vtv[..., None] * Wf1[None, :, :]
                + iou[..., None] * Wf14[None, :, :]
                + o_ij[..., None] * Wf15[None, :, :]
                + o_ji[..., None] * Wf16[None, :, :]
                + fjc[None, :, :] + fi[:, None, :])
        hid = _leaky(base)
        rho_ref[s] = jnp.sum(hid * Wd0[None, :, :], axis=2) + bd


def _head_call(bk, v2, b3, qz, wq, Wv, bv, WfJ, WfI, Wrows, bs, bd):
    grid = (B // bk,)
    kappa, rho = pl.pallas_call(
        functools.partial(_head_body, bk),
        grid=grid,
        in_specs=[
            pl.BlockSpec((bk * K, V_DIM), lambda i: (i, 0)),
            pl.BlockSpec((bk, K, 6), lambda i: (i, 0, 0)),
            pl.BlockSpec((bk, SCORE_DIM), lambda i: (i, 0)),
            pl.BlockSpec((bk, 1), lambda i: (i, 0)),
            pl.BlockSpec((SCORE_DIM, V_DIM), lambda i: (0, 0)),
            pl.BlockSpec((1, SCORE_DIM), lambda i: (0, 0)),
            pl.BlockSpec((HID, 6), lambda i: (0, 0)),
            pl.BlockSpec((HID, 6), lambda i: (0, 0)),
            pl.BlockSpec((7, HID), lambda i: (0, 0)),
            pl.BlockSpec(memory_space=pltpu.SMEM),
            pl.BlockSpec(memory_space=pltpu.SMEM),
        ],
        out_specs=[
            pl.BlockSpec((bk, K), lambda i: (i, 0)),
            pl.BlockSpec((bk, K, K), lambda i: (i, 0, 0)),
        ],
        out_shape=[
            jax.ShapeDtypeStruct((B, K), jnp.float32),
            jax.ShapeDtypeStruct((B, K, K), jnp.float32),
        ],
        scratch_shapes=[pltpu.VMEM((bk * K, bk * K), jnp.float32),
                        pltpu.VMEM((K, K), jnp.float32),
                        pltpu.VMEM((bk * K, bk), jnp.float32)],
    )(v2, b3, qz, wq, Wv, bv, WfJ, WfI, Wrows, bs, bd)
    return kappa, rho


def kernel(v_emb, b, q, embd_table, W_ih, W_hh, b_ih, b_hh, Wv, bv, Wqp, bqp,
           Ws, bs, Wqr, bqr, Wf, bf, Wd, bd):
    WPAD = 384
    table_p, idxt = _pad_call(embd_table, q)
    gather = _make_sc_gather(embd_table.shape[0], WPAD, L * B)
    words = gather(table_p, idxt)

    qz, wq = _gru_call(words, W_ih, W_hh, b_ih, b_hh, Wqp,
                       bqp.reshape(1, -1), Ws, Wqr, bqr.reshape(1, 1))

    WfJ = Wf[:, 2:8]
    WfI = Wf[:, 8:14]
    Wrows = jnp.stack([Wf[:, 0], Wf[:, 1], Wf[:, 14], Wf[:, 15], Wf[:, 16],
                       bf, Wd[0]], axis=0)

    bk = 8
    kappa, rho = _head_call(
        bk, v_emb.reshape(B * K, V_DIM), b, qz, wq,
        Wv, bv.reshape(1, -1), WfJ, WfI, Wrows, bs, bd)
    return kappa, rho


# speedup vs baseline: 1.0205x; 1.0205x over previous
import functools

import jax
import jax.numpy as jnp
from jax import lax
from jax.experimental import pallas as pl
from jax.experimental.pallas import tpu as pltpu
from jax.experimental.pallas import tpu_sc as plsc

B, K, L = 128, 36, 20
WORD_DIM = 300
QUES_DIM = 1024
V_DIM = 2048
SCORE_DIM = 1024
HID = 100


def _leaky(x):
    return jnp.maximum(x, 0.01 * x)


@functools.lru_cache(maxsize=None)
def _make_sc_gather(V, D, N):
    NC, NS = 2, 16
    NW = NC * NS
    n_per_w = N // NW
    mesh = plsc.VectorSubcoreMesh(core_axis_name="c", subcore_axis_name="s",
                                  num_cores=NC, num_subcores=NS)

    @functools.partial(
        pl.kernel,
        out_type=jax.ShapeDtypeStruct((N, D), jnp.float32),
        mesh=mesh,
        scratch_types=[
            pltpu.VMEM((B,), jnp.int32),
            pltpu.VMEM((B, D), jnp.float32),
            pltpu.SemaphoreType.DMA,
        ],
    )
    def gather(table_hbm, idxt_hbm, out_hbm, idx_v, rows_v, sem):
        wid = lax.axis_index("s") * NC + lax.axis_index("c")

        @pl.when(wid < L)
        def _():
            pltpu.sync_copy(idxt_hbm.at[wid], idx_v)
            pltpu.async_copy(table_hbm.at[idx_v], rows_v, sem).wait()
            pltpu.sync_copy(rows_v, out_hbm.at[pl.ds(wid * B, B)])

    return gather


_PAD_ROWS = 2048


def _pad_body(in_ref, q_ref, out_ref, idxt_ref):
    blk = in_ref.shape[0]
    out_ref[...] = jnp.concatenate(
        [in_ref[...], jnp.zeros((blk, 384 - WORD_DIM), jnp.float32)], axis=1)

    @pl.when(pl.program_id(0) == 0)
    def _():
        idxt_ref[...] = jnp.transpose(q_ref[...])


def _pad_call(table, q):
    V = table.shape[0]
    grid = (pl.cdiv(V, _PAD_ROWS),)
    return pl.pallas_call(
        _pad_body,
        grid=grid,
        in_specs=[pl.BlockSpec((_PAD_ROWS, WORD_DIM), lambda i: (i, 0)),
                  pl.BlockSpec((B, L), lambda i: (0, 0))],
        out_specs=[pl.BlockSpec((_PAD_ROWS, 384), lambda i: (i, 0)),
                   pl.BlockSpec((L, B), lambda i: (0, 0))],
        out_shape=[jax.ShapeDtypeStruct((V, 384), jnp.float32),
                   jax.ShapeDtypeStruct((L, B), jnp.int32)],
    )(table, q)


_CHUNK = L // 4


def _gru_body(words_ref, Wih_ref, Whh_ref, bih_ref, bhh_ref, Wqp_ref, bqp_ref,
              Ws_ref, Wqr_ref, bqr_ref, qz_ref, wq_ref, gi_ref):
    Wih = Wih_ref[...].astype(jnp.bfloat16)
    Whh = Whh_ref[...].astype(jnp.bfloat16)
    bih = bih_ref[...]
    bhh = bhh_ref[...]

    def step(t, h):
        gi = gi_ref[pl.ds(t * B, B), :]
        gh = lax.dot_general(h.astype(jnp.bfloat16), Whh, (((1,), (1,)), ((), ())),
                             preferred_element_type=jnp.float32) + bhh
        r = jax.nn.sigmoid(gi[:, :QUES_DIM] + gh[:, :QUES_DIM])
        z = jax.nn.sigmoid(gi[:, QUES_DIM:2 * QUES_DIM] + gh[:, QUES_DIM:2 * QUES_DIM])
        n = jnp.tanh(gi[:, 2 * QUES_DIM:] + r * gh[:, 2 * QUES_DIM:])
        return (1.0 - z) * n + z * h

    h = jnp.zeros((B, QUES_DIM), jnp.float32)
    for part in range(4):
        xs = words_ref[pl.ds(part * _CHUNK * B, _CHUNK * B), :WORD_DIM]
        gi_ref[...] = lax.dot_general(xs.astype(jnp.bfloat16), Wih,
                                      (((1,), (1,)), ((), ())),
                                      preferred_element_type=jnp.float32) + bih
        h = lax.fori_loop(0, _CHUNK, step, h)

    qp = _leaky(lax.dot_general(h, Wqp_ref[...], (((1,), (1,)), ((), ())),
                                preferred_element_type=jnp.float32) + bqp_ref[...])
    qz_ref[...] = qp * Ws_ref[...]
    wq_ref[...] = jnp.sum(h * Wqr_ref[...], axis=1, keepdims=True) + bqr_ref[...]


def _gru_call(words2d, W_ih, W_hh, b_ih, b_hh, Wqp, bqp, Ws, Wqr, bqr):
    return pl.pallas_call(
        _gru_body,
        out_shape=(
            jax.ShapeDtypeStruct((B, SCORE_DIM), jnp.float32),
            jax.ShapeDtypeStruct((B, 1), jnp.float32),
        ),
        scratch_shapes=[pltpu.VMEM((_CHUNK * B, 3 * QUES_DIM), jnp.float32)],
    )(words2d, W_ih, W_hh, b_ih, b_hh, Wqp, bqp, Ws, Wqr, bqr)


def _head_body(bk, v3_ref, b3_ref, qz_ref, wq_ref, Wv_ref, bv_ref,
               WfJ_ref, WfI_ref, Wrows_ref, bs_ref, bd_ref,
               kappa_ref, rho_ref, G_ref, vtv_ref, km_ref):
    v2 = jnp.concatenate([v3_ref[s] for s in range(bk)], axis=0)
    vp = _leaky(lax.dot_general(v2.astype(jnp.bfloat16),
                                Wv_ref[...].astype(jnp.bfloat16),
                                (((1,), (1,)), ((), ())),
                                preferred_element_type=jnp.float32) + bv_ref[...])
    qz = qz_ref[...]
    km_ref[...] = lax.dot_general(vp.astype(jnp.bfloat16),
                                  qz.astype(jnp.bfloat16), (((1,), (1,)), ((), ())),
                                  preferred_element_type=jnp.float32)
    kappa_ref[...] = jnp.concatenate(
        [jnp.transpose(km_ref[pl.ds(s * K, K), s:s + 1]) for s in range(bk)],
        axis=0) + bs_ref[0]

    ssq = jnp.sum(v2 * v2, axis=1, keepdims=True)
    inv = 1.0 / jnp.maximum(jnp.sqrt(ssq), 1e-12)
    nv = (v2 * inv).astype(jnp.bfloat16)
    G_ref[...] = lax.dot_general(nv, nv, (((1,), (1,)), ((), ())),
                                 preferred_element_type=jnp.float32)

    Wf0 = Wrows_ref[0:1, :]
    Wf1 = Wrows_ref[1:2, :]
    Wf14 = Wrows_ref[2:3, :]
    Wf15 = Wrows_ref[3:4, :]
    Wf16 = Wrows_ref[4:5, :]
    bf = Wrows_ref[5:6, :]
    Wd0 = Wrows_ref[6:7, :]
    bd = bd_ref[0]

    for s in range(bk):
        vtv_ref[...] = G_ref[pl.ds(s * K, K), pl.ds(s * K, K)]
        vtv = vtv_ref[...]

        boxes = b3_ref[s]
        x1 = boxes[:, 0:1]
        y1 = boxes[:, 1:2]
        x2 = boxes[:, 2:3]
        y2 = boxes[:, 3:4]
        area = (x2 - x1) * (y2 - y1)

        def bi(col):
            return jnp.broadcast_to(col, (K, K))

        def bj(col):
            return jnp.broadcast_to(jnp.transpose(col), (K, K))

        rl = jnp.maximum(bj(x1), bi(x1))
        dt = jnp.maximum(bj(y1), bi(y1))
        lr = jnp.minimum(bj(x2), bi(x2))
        td = jnp.minimum(bj(y2), bi(y2))
        olap = jnp.maximum(0.0, lr - rl) * jnp.maximum(0.0, td - dt)
        area_i = bi(area)
        area_j = bj(area)
        iou = olap / (area_i + area_j - olap)
        o_ij = olap / area_j
        o_ji = olap / area_i

        fj = lax.dot_general(boxes, WfJ_ref[...], (((1,), (1,)), ((), ())),
                             preferred_element_type=jnp.float32)
        fi = lax.dot_general(boxes, WfI_ref[...], (((1,), (1,)), ((), ())),
                             preferred_element_type=jnp.float32)
        wq_s = wq_ref[s:s + 1, :]
        fjc = fj + wq_s * Wf0 + bf

        base = (vtv[..., None] * Wf1[None, :, :]
                + iou[..., None] * Wf14[None, :, :]
                + o_ij[..., None] * Wf15[None, :, :]
                + o_ji[..., None] * Wf16[None, :, :]
                + fjc[None, :, :] + fi[:, None, :])
        hid = _leaky(base)
        rho_ref[s] = jnp.sum(hid * Wd0[None, :, :], axis=2) + bd


def _head_call(bk, v2, b3, qz, wq, Wv, bv, WfJ, WfI, Wrows, bs, bd):
    grid = (B // bk,)
    kappa, rho = pl.pallas_call(
        functools.partial(_head_body, bk),
        grid=grid,
        in_specs=[
            pl.BlockSpec((bk, K, V_DIM), lambda i: (i, 0, 0)),
            pl.BlockSpec((bk, K, 6), lambda i: (i, 0, 0)),
            pl.BlockSpec((bk, SCORE_DIM), lambda i: (i, 0)),
            pl.BlockSpec((bk, 1), lambda i: (i, 0)),
            pl.BlockSpec((SCORE_DIM, V_DIM), lambda i: (0, 0)),
            pl.BlockSpec((1, SCORE_DIM), lambda i: (0, 0)),
            pl.BlockSpec((HID, 6), lambda i: (0, 0)),
            pl.BlockSpec((HID, 6), lambda i: (0, 0)),
            pl.BlockSpec((7, HID), lambda i: (0, 0)),
            pl.BlockSpec(memory_space=pltpu.SMEM),
            pl.BlockSpec(memory_space=pltpu.SMEM),
        ],
        out_specs=[
            pl.BlockSpec((bk, K), lambda i: (i, 0)),
            pl.BlockSpec((bk, K, K), lambda i: (i, 0, 0)),
        ],
        out_shape=[
            jax.ShapeDtypeStruct((B, K), jnp.float32),
            jax.ShapeDtypeStruct((B, K, K), jnp.float32),
        ],
        scratch_shapes=[pltpu.VMEM((bk * K, bk * K), jnp.float32),
                        pltpu.VMEM((K, K), jnp.float32),
                        pltpu.VMEM((bk * K, bk), jnp.float32)],
    )(v2, b3, qz, wq, Wv, bv, WfJ, WfI, Wrows, bs, bd)
    return kappa, rho


def kernel(v_emb, b, q, embd_table, W_ih, W_hh, b_ih, b_hh, Wv, bv, Wqp, bqp,
           Ws, bs, Wqr, bqr, Wf, bf, Wd, bd):
    WPAD = 384
    table_p, idxt = _pad_call(embd_table, q)
    gather = _make_sc_gather(embd_table.shape[0], WPAD, L * B)
    words = gather(table_p, idxt)

    qz, wq = _gru_call(words, W_ih, W_hh, b_ih, b_hh, Wqp,
                       bqp.reshape(1, -1), Ws, Wqr, bqr.reshape(1, 1))

    WfJ = Wf[:, 2:8]
    WfI = Wf[:, 8:14]
    Wrows = jnp.stack([Wf[:, 0], Wf[:, 1], Wf[:, 14], Wf[:, 15], Wf[:, 16],
                       bf, Wd[0]], axis=0)

    bk = 8
    kappa, rho = _head_call(
        bk, v_emb, b, qz, wq,
        Wv, bv.reshape(1, -1), WfJ, WfI, Wrows, bs, bd)
    return kappa, rho
